# baseline (speedup 1.0000x reference)
import functools

import jax
import jax.numpy as jnp
from jax import lax
from jax.experimental import pallas as pl
from jax.experimental.pallas import tpu as pltpu
from jax.experimental.pallas import tpu_sc as plsc

NC = 2
NS = 16
NW = NC * NS
CHUNK = 128


@functools.lru_cache(maxsize=None)
def _make_gather(B, D):
    b_per_w = B // NW
    n_chunks = b_per_w // CHUNK
    mesh = plsc.VectorSubcoreMesh(
        core_axis_name="c", subcore_axis_name="s", num_cores=NC, num_subcores=NS
    )

    @functools.partial(
        pl.kernel,
        out_type=[jax.ShapeDtypeStruct((B, D), jnp.float32) for _ in range(4)],
        mesh=mesh,
        compiler_params=pltpu.CompilerParams(use_tc_tiling_on_sc=False),
        scratch_types=[
            pltpu.VMEM((b_per_w,), jnp.int32),
            pltpu.VMEM((b_per_w,), jnp.int32),
            pltpu.VMEM((b_per_w, D), jnp.float32),
            pltpu.VMEM((b_per_w, D), jnp.float32),
            pltpu.VMEM((b_per_w, D), jnp.float32),
            pltpu.VMEM((b_per_w, D), jnp.float32),
            pltpu.SemaphoreType.DMA,
            pltpu.SemaphoreType.DMA,
            pltpu.SemaphoreType.DMA,
            pltpu.SemaphoreType.DMA,
        ],
    )
    def gather_kernel(uids_hbm, iids_hbm, umf_hbm, imf_hbm, umlp_hbm, imlp_hbm,
                      out_umf, out_imf, out_umlp, out_imlp,
                      uidx_v, iidx_v, r0, r1, r2, r3, s0, s1, s2, s3):
        wid = lax.axis_index("s") * NC + lax.axis_index("c")
        base = wid * b_per_w
        pltpu.sync_copy(uids_hbm.at[pl.ds(base, b_per_w)], uidx_v)
        pltpu.sync_copy(iids_hbm.at[pl.ds(base, b_per_w)], iidx_v)
        copies = []
        for k in range(n_chunks):
            sl = pl.ds(k * CHUNK, CHUNK)
            copies.append(pltpu.async_copy(umf_hbm.at[uidx_v.at[sl]], r0.at[sl], s0))
            copies.append(pltpu.async_copy(imf_hbm.at[iidx_v.at[sl]], r1.at[sl], s1))
            copies.append(pltpu.async_copy(umlp_hbm.at[uidx_v.at[sl]], r2.at[sl], s2))
            copies.append(pltpu.async_copy(imlp_hbm.at[iidx_v.at[sl]], r3.at[sl], s3))
        for c in copies:
            c.wait()
        out_sl = pl.ds(base, b_per_w)
        pltpu.sync_copy(r0, out_umf.at[out_sl])
        pltpu.sync_copy(r1, out_imf.at[out_sl])
        pltpu.sync_copy(r2, out_umlp.at[out_sl])
        pltpu.sync_copy(r3, out_imlp.at[out_sl])

    return gather_kernel


def _mlp_body(umf_ref, imf_ref, umlp_ref, imlp_ref,
              w1_ref, b1_ref, w2_ref, b2_ref, wp_ref, bp_ref, out_ref):
    mf = umf_ref[...] * imf_ref[...]
    w1 = w1_ref[...]
    dn = (((1,), (1,)), ((), ()))
    h1 = (lax.dot_general(umlp_ref[...], w1[:, :32], dn,
                          preferred_element_type=jnp.float32)
          + lax.dot_general(imlp_ref[...], w1[:, 32:], dn,
                            preferred_element_type=jnp.float32)
          + b1_ref[...])
    h1 = jnp.maximum(h1, 0.0)
    h2 = lax.dot_general(h1, w2_ref[...], dn,
                         preferred_element_type=jnp.float32) + b2_ref[...]
    h2 = jnp.maximum(h2, 0.0)
    wp = wp_ref[...]
    logit = (lax.dot_general(mf, wp[:, :32], dn,
                             preferred_element_type=jnp.float32)
             + lax.dot_general(h2, wp[:, 32:], dn,
                               preferred_element_type=jnp.float32)
             + bp_ref[...])
    out_ref[...] = jax.nn.sigmoid(logit) * 5.0


@functools.lru_cache(maxsize=None)
def _make_mlp(B, blk, interpret=False):
    n_blocks = B // blk
    return pl.pallas_call(
        _mlp_body,
        grid=(n_blocks,),
        in_specs=[
            pl.BlockSpec((blk, 32), lambda i: (i, 0)),
            pl.BlockSpec((blk, 32), lambda i: (i, 0)),
            pl.BlockSpec((blk, 32), lambda i: (i, 0)),
            pl.BlockSpec((blk, 32), lambda i: (i, 0)),
            pl.BlockSpec((64, 64), lambda i: (0, 0)),
            pl.BlockSpec((1, 64), lambda i: (0, 0)),
            pl.BlockSpec((32, 64), lambda i: (0, 0)),
            pl.BlockSpec((1, 32), lambda i: (0, 0)),
            pl.BlockSpec((1, 64), lambda i: (0, 0)),
            pl.BlockSpec((1, 1), lambda i: (0, 0)),
        ],
        out_specs=pl.BlockSpec((blk, 1), lambda i: (i, 0)),
        out_shape=jax.ShapeDtypeStruct((B, 1), jnp.float32),
        interpret=interpret,
    )


def kernel(user_ids, item_ids, user_mf_emb, item_mf_emb, user_mlp_emb,
           item_mlp_emb, W1, b1, W2, b2, Wp, bp):
    B = user_ids.shape[0]
    D = user_mf_emb.shape[1]
    gather = _make_gather(B, D)
    umf, imf, umlp, imlp = gather(user_ids, item_ids, user_mf_emb, item_mf_emb,
                                  user_mlp_emb, item_mlp_emb)
    mlp = _make_mlp(B, 2048)
    return mlp(umf, imf, umlp, imlp,
               W1, b1.reshape(1, -1), W2, b2.reshape(1, -1),
               Wp, bp.reshape(1, 1))
